# baseline (speedup 1.0000x reference)
import jax
import jax.numpy as jnp
from jax.experimental import pallas as pl


def _pos_body(xe_ref, ye_ref, o_ref):
    f = xe_ref.shape[1]
    h = ye_ref.shape[0]
    w = xe_ref.shape[0]
    xet = xe_ref[...].T
    yet = ye_ref[...].T
    o_ref[0, :f] = jnp.broadcast_to(xet[:, None, :], (f, h, w))
    o_ref[0, f:] = jnp.broadcast_to(yet[:, :, None], (f, h, w))


def kernel(x, xenc, yenc):
    b = x.shape[0]
    h, w = x.shape[-2], x.shape[-1]
    f = xenc.shape[1]
    out = pl.pallas_call(
        _pos_body,
        grid=(b,),
        in_specs=[
            pl.BlockSpec((w, f), lambda i: (0, 0)),
            pl.BlockSpec((h, f), lambda i: (0, 0)),
        ],
        out_specs=pl.BlockSpec((1, 2 * f, h, w), lambda i: (i, 0, 0, 0)),
        out_shape=jax.ShapeDtypeStruct((b, 2 * f, h, w), xenc.dtype),
    )(xenc[:w], yenc[:h])
    return out


# speedup vs baseline: 2.6669x; 2.6669x over previous
import jax
import jax.numpy as jnp
from jax import lax
from jax.experimental import pallas as pl
from jax.experimental.pallas import tpu as pltpu


def _make_body(b, f, h, w):
    hw = h * w

    def body(xe_ref, ye_ref, o_ref, scratch_ref, sem):
        k = lax.broadcasted_iota(jnp.int32, (w, hw), 1)
        r = lax.broadcasted_iota(jnp.int32, (w, hw), 0)
        sel_x = (k % w == r).astype(jnp.float32)
        sel_y = (k // w == r).astype(jnp.float32)
        dn = (((0,), (0,)), ((), ()))
        scratch_ref[:f] = lax.dot_general(
            xe_ref[...], sel_x, dn, preferred_element_type=jnp.float32)
        scratch_ref[f:] = lax.dot_general(
            ye_ref[...], sel_y, dn, preferred_element_type=jnp.float32)
        for i in range(b):
            pltpu.make_async_copy(scratch_ref, o_ref.at[i], sem).start()
        for i in range(b):
            pltpu.make_async_copy(scratch_ref, o_ref.at[i], sem).wait()

    return body


def kernel(x, xenc, yenc):
    b = x.shape[0]
    h, w = x.shape[-2], x.shape[-1]
    f = xenc.shape[1]
    out = pl.pallas_call(
        _make_body(b, f, h, w),
        in_specs=[
            pl.BlockSpec(memory_space=pltpu.MemorySpace.VMEM),
            pl.BlockSpec(memory_space=pltpu.MemorySpace.VMEM),
        ],
        out_specs=pl.BlockSpec(memory_space=pltpu.MemorySpace.HBM),
        out_shape=jax.ShapeDtypeStruct((b, 2 * f, h * w), jnp.float32),
        scratch_shapes=[
            pltpu.VMEM((2 * f, h * w), jnp.float32),
            pltpu.SemaphoreType.DMA,
        ],
    )(xenc[:w], yenc[:h])
    return out.reshape(b, 2 * f, h, w)


# speedup vs baseline: 2.7001x; 1.0124x over previous
import jax
import jax.numpy as jnp
from jax import lax
from jax.experimental import pallas as pl
from jax.experimental.pallas import tpu as pltpu


def _make_body(b, f, h, w):
    hw = h * w

    def body(xe_ref, ye_ref, o_ref, scratch_ref, sem):
        k = lax.broadcasted_iota(jnp.int32, (w, hw), 1)
        r = lax.broadcasted_iota(jnp.int32, (w, hw), 0)
        sel_x = (k % w == r).astype(jnp.float32)
        sel_y = (k // w == r).astype(jnp.float32)
        dn = (((0,), (0,)), ((), ()))
        scratch_ref[:f] = lax.dot_general(
            xe_ref[...], sel_x, dn, preferred_element_type=jnp.float32)
        scratch_ref[f:] = lax.dot_general(
            ye_ref[...], sel_y, dn, preferred_element_type=jnp.float32)
        n_sem = sem.shape[0]
        for i in range(b):
            pltpu.make_async_copy(scratch_ref, o_ref.at[i],
                                  sem.at[i % n_sem]).start()
        for i in range(b):
            pltpu.make_async_copy(scratch_ref, o_ref.at[i],
                                  sem.at[i % n_sem]).wait()

    return body


def kernel(x, xenc, yenc):
    b = x.shape[0]
    h, w = x.shape[-2], x.shape[-1]
    f = xenc.shape[1]
    out = pl.pallas_call(
        _make_body(b, f, h, w),
        in_specs=[
            pl.BlockSpec(memory_space=pltpu.MemorySpace.VMEM),
            pl.BlockSpec(memory_space=pltpu.MemorySpace.VMEM),
        ],
        out_specs=pl.BlockSpec(memory_space=pltpu.MemorySpace.HBM),
        out_shape=jax.ShapeDtypeStruct((b, 2 * f, h * w), jnp.float32),
        scratch_shapes=[
            pltpu.VMEM((2 * f, h * w), jnp.float32),
            pltpu.SemaphoreType.DMA((8,)),
        ],
    )(xenc[:w], yenc[:h])
    return out.reshape(b, 2 * f, h, w)
